# baseline (speedup 1.0000x reference)
import functools

import jax
import jax.numpy as jnp
from jax.experimental import pallas as pl
from jax.experimental.pallas import tpu as pltpu

_E = 8
_H = 1024
_I = 2048
_G = 128
_T = 256

_NG = _H // _G
_NG2 = _I // _G


def _moe_body(hs_ref, logits_ref, w13a_ref, w13b_ref, s13_ref,
              w2a_ref, w2b_ref, s2_ref,
              p13_ref, p2_ref, out_ref, w1s, w3s, w2s):
    e = pl.program_id(0)

    @pl.when(e == 0)
    def _init():
        out_ref[...] = jnp.zeros_like(out_ref)

    logits = logits_ref[...]
    iota = jax.lax.broadcasted_iota(jnp.int32, logits.shape, 1)
    m1 = jnp.max(logits, axis=1, keepdims=True)
    a1 = jnp.min(jnp.where(logits == m1, iota, _E), axis=1, keepdims=True)
    masked = jnp.where(iota == a1, -jnp.inf, logits)
    m2 = jnp.max(masked, axis=1, keepdims=True)
    a2 = jnp.min(jnp.where(masked == m2, iota, _E), axis=1, keepdims=True)
    p_top = jax.nn.sigmoid(m1 - m2)
    p_snd = jax.nn.sigmoid(m2 - m1)
    fscale = (jnp.where(a1 == e, p_top, 0.0)
              + jnp.where(a2 == e, p_snd, 0.0))

    p13 = p13_ref[0, 0, 0]
    p2 = p2_ref[0, 0, 0]

    for g in range(_NG):
        sl = slice(g * _G, (g + 1) * _G)
        w13_ref = w13a_ref if g < _NG // 2 else w13b_ref
        hsl = slice((g % (_NG // 2)) * _G, (g % (_NG // 2) + 1) * _G)
        w1s[sl, :] = (w13_ref[0, hsl, :_I].astype(jnp.float32)
                      * s13_ref[0, g, :_I][None, :])
        w3s[sl, :] = (w13_ref[0, hsl, _I:].astype(jnp.float32)
                      * s13_ref[0, g, _I:][None, :])
    for g in range(_NG2):
        sl = slice(g * _G, (g + 1) * _G)
        w2_ref = w2a_ref if g < _NG2 // 2 else w2b_ref
        isl = slice((g % (_NG2 // 2)) * _G, (g % (_NG2 // 2) + 1) * _G)
        w2s[sl, :] = (w2_ref[0, isl, :].astype(jnp.float32)
                      * s2_ref[0, g, :][None, :])

    dot = functools.partial(
        jax.lax.dot_general,
        dimension_numbers=(((1,), (0,)), ((), ())),
        preferred_element_type=jnp.float32,
    )

    aq = (jnp.clip(hs_ref[...] / p13, -448.0, 448.0)
          .astype(jnp.float8_e4m3fn).astype(jnp.float32))
    fc1 = dot(aq, w1s[...]) * p13
    gate = dot(aq, w3s[...]) * p13
    h2 = fc1 * (gate * jax.nn.sigmoid(gate))

    aq2 = (jnp.clip(h2 / p2, -448.0, 448.0)
           .astype(jnp.float8_e4m3fn).astype(jnp.float32))
    fc2 = dot(aq2, w2s[...]) * p2

    out_ref[...] += fc2 * fscale


def kernel(hidden_states, router_logits, W13, W2, S13, S2, P13, P2):
    hs = hidden_states.reshape(-1, _H)
    p13b = jnp.broadcast_to(P13[:, None, None], (_E, 8, 128))
    p2b = jnp.broadcast_to(P2[:, None, None], (_E, 8, 128))

    out = pl.pallas_call(
        _moe_body,
        grid=(_E,),
        in_specs=[
            pl.BlockSpec((_T, _H), lambda e: (0, 0)),
            pl.BlockSpec((_T, _E), lambda e: (0, 0)),
            pl.BlockSpec((1, _H // 2, 2 * _I), lambda e: (e, 0, 0)),
            pl.BlockSpec((1, _H // 2, 2 * _I), lambda e: (e, 1, 0)),
            pl.BlockSpec((1, _NG, 2 * _I), lambda e: (e, 0, 0)),
            pl.BlockSpec((1, _I // 2, _H), lambda e: (e, 0, 0)),
            pl.BlockSpec((1, _I // 2, _H), lambda e: (e, 1, 0)),
            pl.BlockSpec((1, _NG2, _H), lambda e: (e, 0, 0)),
            pl.BlockSpec((1, 8, 128), lambda e: (e, 0, 0)),
            pl.BlockSpec((1, 8, 128), lambda e: (e, 0, 0)),
        ],
        out_specs=pl.BlockSpec((_T, _H), lambda e: (0, 0)),
        out_shape=jax.ShapeDtypeStruct((_T, _H), jnp.float32),
        scratch_shapes=[
            pltpu.VMEM((_H, _I), jnp.float32),
            pltpu.VMEM((_H, _I), jnp.float32),
            pltpu.VMEM((_I, _H), jnp.float32),
        ],
        compiler_params=pltpu.CompilerParams(
            dimension_semantics=("arbitrary",)),
    )(hs, router_logits, W13, W13, S13, W2, W2, S2, p13b, p2b)

    return out.reshape(hidden_states.shape)


# speedup vs baseline: 1.0537x; 1.0537x over previous
import functools

import jax
import jax.numpy as jnp
from jax.experimental import pallas as pl
from jax.experimental.pallas import tpu as pltpu

_E = 8
_H = 1024
_I = 2048
_G = 128
_T = 256

_NG = _H // _G
_NG2 = _I // _G


def _moe_body(hs_ref, logits_ref, w13a_ref, w13b_ref, s13_ref,
              w2a_ref, w2b_ref, s2_ref,
              p13_ref, p2_ref, out_ref, w1s, w3s, w2s):
    e = pl.program_id(0)

    @pl.when(e == 0)
    def _init():
        out_ref[...] = jnp.zeros_like(out_ref)

    logits = logits_ref[...]
    iota = jax.lax.broadcasted_iota(jnp.int32, logits.shape, 1)
    m1 = jnp.max(logits, axis=1, keepdims=True)
    a1 = jnp.min(jnp.where(logits == m1, iota, _E), axis=1, keepdims=True)
    masked = jnp.where(iota == a1, -jnp.inf, logits)
    m2 = jnp.max(masked, axis=1, keepdims=True)
    a2 = jnp.min(jnp.where(masked == m2, iota, _E), axis=1, keepdims=True)
    p_top = jax.nn.sigmoid(m1 - m2)
    p_snd = jax.nn.sigmoid(m2 - m1)
    fscale = (jnp.where(a1 == e, p_top, 0.0)
              + jnp.where(a2 == e, p_snd, 0.0))

    p13 = p13_ref[e]
    p2 = p2_ref[e]

    for g in range(_NG):
        sl = slice(g * _G, (g + 1) * _G)
        w13_ref = w13a_ref if g < _NG // 2 else w13b_ref
        hsl = slice((g % (_NG // 2)) * _G, (g % (_NG // 2) + 1) * _G)
        w1s[sl, :] = (w13_ref[0, hsl, :_I].astype(jnp.float32)
                      * s13_ref[0, g, :_I][None, :])
        w3s[sl, :] = (w13_ref[0, hsl, _I:].astype(jnp.float32)
                      * s13_ref[0, g, _I:][None, :])
    for g in range(_NG2):
        sl = slice(g * _G, (g + 1) * _G)
        w2_ref = w2a_ref if g < _NG2 // 2 else w2b_ref
        isl = slice((g % (_NG2 // 2)) * _G, (g % (_NG2 // 2) + 1) * _G)
        w2s[sl, :] = (w2_ref[0, isl, :].astype(jnp.float32)
                      * s2_ref[0, g, :][None, :])

    dot = functools.partial(
        jax.lax.dot_general,
        dimension_numbers=(((1,), (0,)), ((), ())),
        preferred_element_type=jnp.float32,
    )

    aq = (jnp.clip(hs_ref[...] / p13, -448.0, 448.0)
          .astype(jnp.float8_e4m3fn).astype(jnp.float32))
    fc1 = dot(aq, w1s[...]) * p13
    gate = dot(aq, w3s[...]) * p13
    h2 = fc1 * (gate * jax.nn.sigmoid(gate))

    aq2 = (jnp.clip(h2 / p2, -448.0, 448.0)
           .astype(jnp.float8_e4m3fn).astype(jnp.float32))
    fc2 = dot(aq2, w2s[...]) * p2

    out_ref[...] += fc2 * fscale


def kernel(hidden_states, router_logits, W13, W2, S13, S2, P13, P2):
    hs = hidden_states.reshape(-1, _H)

    out = pl.pallas_call(
        _moe_body,
        grid=(_E,),
        in_specs=[
            pl.BlockSpec((_T, _H), lambda e: (0, 0)),
            pl.BlockSpec((_T, _E), lambda e: (0, 0)),
            pl.BlockSpec((1, _H // 2, 2 * _I), lambda e: (e, 0, 0)),
            pl.BlockSpec((1, _H // 2, 2 * _I), lambda e: (e, 1, 0)),
            pl.BlockSpec((1, _NG, 2 * _I), lambda e: (e, 0, 0)),
            pl.BlockSpec((1, _I // 2, _H), lambda e: (e, 0, 0)),
            pl.BlockSpec((1, _I // 2, _H), lambda e: (e, 1, 0)),
            pl.BlockSpec((1, _NG2, _H), lambda e: (e, 0, 0)),
            pl.BlockSpec(memory_space=pltpu.SMEM),
            pl.BlockSpec(memory_space=pltpu.SMEM),
        ],
        out_specs=pl.BlockSpec((_T, _H), lambda e: (0, 0)),
        out_shape=jax.ShapeDtypeStruct((_T, _H), jnp.float32),
        scratch_shapes=[
            pltpu.VMEM((_H, _I), jnp.float32),
            pltpu.VMEM((_H, _I), jnp.float32),
            pltpu.VMEM((_I, _H), jnp.float32),
        ],
        compiler_params=pltpu.CompilerParams(
            dimension_semantics=("arbitrary",)),
    )(hs, router_logits, W13, W13, S13, W2, W2, S2, P13, P2)

    return out.reshape(hidden_states.shape)


# speedup vs baseline: 1.0541x; 1.0004x over previous
import functools

import jax
import jax.numpy as jnp
from jax.experimental import pallas as pl
from jax.experimental.pallas import tpu as pltpu

_E = 8
_H = 1024
_I = 2048
_G = 128
_T = 256

_NG = _H // _G
_NG2 = _I // _G


def _moe_body(hs_ref, logits_ref, w13_any, s13_ref, w2_any, s2_ref,
              p13_ref, p2_ref, out_ref, w13b, w2b, w1s, w3s, w2s,
              sem13, sem2):
    e = pl.program_id(0)
    slot = jax.lax.rem(e, 2)
    nslot = 1 - slot

    @pl.when(e == 0)
    def _init():
        out_ref[...] = jnp.zeros_like(out_ref)
        pltpu.make_async_copy(w13_any.at[0], w13b.at[0], sem13.at[0]).start()
        pltpu.make_async_copy(w2_any.at[0], w2b.at[0], sem2.at[0]).start()

    @pl.when(e + 1 < _E)
    def _prefetch():
        pltpu.make_async_copy(
            w13_any.at[e + 1], w13b.at[nslot], sem13.at[nslot]).start()
        pltpu.make_async_copy(
            w2_any.at[e + 1], w2b.at[nslot], sem2.at[nslot]).start()

    logits = logits_ref[...]
    iota = jax.lax.broadcasted_iota(jnp.int32, logits.shape, 1)
    m1 = jnp.max(logits, axis=1, keepdims=True)
    a1 = jnp.min(jnp.where(logits == m1, iota, _E), axis=1, keepdims=True)
    masked = jnp.where(iota == a1, -jnp.inf, logits)
    m2 = jnp.max(masked, axis=1, keepdims=True)
    a2 = jnp.min(jnp.where(masked == m2, iota, _E), axis=1, keepdims=True)
    p_top = jax.nn.sigmoid(m1 - m2)
    p_snd = jax.nn.sigmoid(m2 - m1)
    fscale = (jnp.where(a1 == e, p_top, 0.0)
              + jnp.where(a2 == e, p_snd, 0.0))

    p13 = p13_ref[e]
    p2 = p2_ref[e]

    pltpu.make_async_copy(w13_any.at[e], w13b.at[slot], sem13.at[slot]).wait()
    pltpu.make_async_copy(w2_any.at[e], w2b.at[slot], sem2.at[slot]).wait()

    for g in range(_NG):
        sl = slice(g * _G, (g + 1) * _G)
        w1s[sl, :] = (w13b[slot, sl, :_I].astype(jnp.float32)
                      * s13_ref[0, g, :_I][None, :])
        w3s[sl, :] = (w13b[slot, sl, _I:].astype(jnp.float32)
                      * s13_ref[0, g, _I:][None, :])
    for g in range(_NG2):
        sl = slice(g * _G, (g + 1) * _G)
        w2s[sl, :] = (w2b[slot, sl, :].astype(jnp.float32)
                      * s2_ref[0, g, :][None, :])

    dot = functools.partial(
        jax.lax.dot_general,
        dimension_numbers=(((1,), (0,)), ((), ())),
        preferred_element_type=jnp.float32,
    )

    aq = (jnp.clip(hs_ref[...] / p13, -448.0, 448.0)
          .astype(jnp.float8_e4m3fn).astype(jnp.float32))
    fc1 = dot(aq, w1s[...]) * p13
    gate = dot(aq, w3s[...]) * p13
    h2 = fc1 * (gate * jax.nn.sigmoid(gate))

    aq2 = (jnp.clip(h2 / p2, -448.0, 448.0)
           .astype(jnp.float8_e4m3fn).astype(jnp.float32))
    fc2 = dot(aq2, w2s[...]) * p2

    out_ref[...] += fc2 * fscale


def kernel(hidden_states, router_logits, W13, W2, S13, S2, P13, P2):
    hs = hidden_states.reshape(-1, _H)

    out = pl.pallas_call(
        _moe_body,
        grid=(_E,),
        in_specs=[
            pl.BlockSpec((_T, _H), lambda e: (0, 0)),
            pl.BlockSpec((_T, _E), lambda e: (0, 0)),
            pl.BlockSpec(memory_space=pltpu.MemorySpace.HBM),
            pl.BlockSpec((1, _NG, 2 * _I), lambda e: (e, 0, 0)),
            pl.BlockSpec(memory_space=pltpu.MemorySpace.HBM),
            pl.BlockSpec((1, _NG2, _H), lambda e: (e, 0, 0)),
            pl.BlockSpec(memory_space=pltpu.SMEM),
            pl.BlockSpec(memory_space=pltpu.SMEM),
        ],
        out_specs=pl.BlockSpec((_T, _H), lambda e: (0, 0)),
        out_shape=jax.ShapeDtypeStruct((_T, _H), jnp.float32),
        scratch_shapes=[
            pltpu.VMEM((2, _H, 2 * _I), jnp.int8),
            pltpu.VMEM((2, _I, _H), jnp.int8),
            pltpu.VMEM((_H, _I), jnp.float32),
            pltpu.VMEM((_H, _I), jnp.float32),
            pltpu.VMEM((_I, _H), jnp.float32),
            pltpu.SemaphoreType.DMA((2,)),
            pltpu.SemaphoreType.DMA((2,)),
        ],
        compiler_params=pltpu.CompilerParams(
            dimension_semantics=("arbitrary",)),
    )(hs, router_logits, W13, S13, W2, S2, P13, P2)

    return out.reshape(hidden_states.shape)
